# baseline (speedup 1.0000x reference)
import jax
import jax.numpy as jnp
import numpy as np
from jax import lax
from jax.experimental import pallas as pl
from jax.experimental.pallas import tpu as pltpu
from jax.experimental.pallas import tpu_sc as plsc

_N = 4096
_K = 17
_KPAD = 32
_NC = 2
_NS = 16
_NW = _NC * _NS
_QPW = _N // _NW
_NBLK = _N // 16
_BIG = np.float32(1e30)


def _sc_body(kx_h, ky_h, kz_h, cell_h, oidx_h, od2_h,
             kx, ky, kz, cellv, d2, lvl1, oidxv, od2v):
    cid = lax.axis_index("c")
    sid = lax.axis_index("s")
    wid = sid * _NC + cid
    base = wid * _QPW
    pltpu.sync_copy(kx_h, kx)
    pltpu.sync_copy(ky_h, ky)
    pltpu.sync_copy(kz_h, kz)
    pltpu.sync_copy(cell_h, cellv)
    lanes = lax.iota(jnp.int32, 16)
    m0 = lanes == 0
    zf = jnp.zeros((16,), jnp.float32)
    zi = jnp.zeros((16,), jnp.int32)
    cx = cellv[pl.ds(0, 16)]
    cy = cellv[pl.ds(16, 16)]
    cz = cellv[pl.ds(32, 16)]
    one = jnp.float32(1.0)

    def per_query(q, _):
        i = base + q
        ti = i % _NBLK
        li = i // _NBLK
        qsplat = zi + (ti * 16 + li)
        qx = plsc.load_gather(kx, [qsplat])
        qy = plsc.load_gather(ky, [qsplat])
        qz = plsc.load_gather(kz, [qsplat])

        def over_sub(b, G):
            def over_row(o, msub):
                t = (b * 16 + o) * 16
                dx = qx - kx[pl.ds(t, 16)]
                ax = jnp.abs(dx)
                wx = jnp.minimum(ax, one - ax) * cx
                acc = wx * wx
                dy = qy - ky[pl.ds(t, 16)]
                ay = jnp.abs(dy)
                wy = jnp.minimum(ay, one - ay) * cy
                acc = acc + wy * wy
                dz = qz - kz[pl.ds(t, 16)]
                az = jnp.abs(dz)
                wz = jnp.minimum(az, one - az) * cz
                acc = acc + wz * wz
                d2[pl.ds(t, 16)] = acc
                return jnp.minimum(msub, acc)

            msub = lax.fori_loop(0, 16, over_row, zf + _BIG)
            lvl1[pl.ds(b * 16, 16)] = msub
            return jnp.minimum(G, msub)

        G = lax.fori_loop(0, 16, over_sub, zf + _BIG)

        bi = ti // 16
        oi = ti % 16
        li_v = zi + li
        plsc.store_scatter(d2, [qsplat], zf + _BIG, mask=m0)
        vals = plsc.load_gather(d2, [(zi + bi * 16 + lanes) * 16 + li_v])
        mn = jnp.min(vals)
        sub = plsc.load_gather(lvl1, [lanes * 16 + li_v])
        subnew = jnp.where(lanes == bi, mn, sub)
        plsc.store_scatter(lvl1, [lanes * 16 + li_v], subnew)
        G = jnp.where(lanes == li_v, jnp.min(subnew), G)

        def extract(k, G):
            gv = jnp.min(G)
            lsel = plsc.all_reduce_ffs(G == gv)
            sub = plsc.load_gather(lvl1, [lanes * 16 + lsel])
            bsel = plsc.all_reduce_ffs(sub == gv)
            vals = plsc.load_gather(d2, [(bsel * 16 + lanes) * 16 + lsel])
            osel = plsc.all_reduce_ffs(vals == gv)
            e = lsel * 256 + bsel * 16 + osel
            slot = zi + q * _KPAD + k
            plsc.store_scatter(oidxv, [slot], e, mask=m0)
            plsc.store_scatter(od2v, [slot], zf + gv, mask=m0)
            plsc.store_scatter(d2, [(bsel * 16 + osel) * 16 + lsel],
                               zf + _BIG, mask=m0)
            vals2 = jnp.where(lanes == osel, _BIG, vals)
            mn = jnp.min(vals2)
            subnew = jnp.where(lanes == bsel, mn, sub)
            plsc.store_scatter(lvl1, [lanes * 16 + lsel], subnew)
            return jnp.where(lanes == lsel, jnp.min(subnew), G)

        lax.fori_loop(0, _K, extract, G)
        return 0

    lax.fori_loop(0, _QPW, per_query, 0)
    pltpu.sync_copy(oidxv, oidx_h.at[pl.ds(base * _KPAD, _QPW * _KPAD)])
    pltpu.sync_copy(od2v, od2_h.at[pl.ds(base * _KPAD, _QPW * _KPAD)])


def kernel(pos, cell):
    n = pos.shape[0]
    frac = pos / cell
    kx = frac[:, 0].reshape(16, _NBLK).T.reshape(-1)
    ky = frac[:, 1].reshape(16, _NBLK).T.reshape(-1)
    kz = frac[:, 2].reshape(16, _NBLK).T.reshape(-1)
    cellvec = jnp.repeat(cell, 16)

    f = pl.kernel(
        _sc_body,
        out_type=[
            jax.ShapeDtypeStruct((n * _KPAD,), jnp.int32),
            jax.ShapeDtypeStruct((n * _KPAD,), jnp.float32),
        ],
        mesh=plsc.VectorSubcoreMesh(core_axis_name="c", subcore_axis_name="s"),
        compiler_params=pltpu.CompilerParams(needs_layout_passes=False),
        scratch_types=[
            pltpu.VMEM((_N,), jnp.float32),
            pltpu.VMEM((_N,), jnp.float32),
            pltpu.VMEM((_N,), jnp.float32),
            pltpu.VMEM((48,), jnp.float32),
            pltpu.VMEM((_N,), jnp.float32),
            pltpu.VMEM((256,), jnp.float32),
            pltpu.VMEM((_QPW * _KPAD,), jnp.int32),
            pltpu.VMEM((_QPW * _KPAD,), jnp.float32),
        ],
    )
    oidx, od2 = f(kx, ky, kz, cellvec)

    idx = oidx.reshape(n, _KPAD)[:, :_K]
    d2 = od2.reshape(n, _KPAD)[:, :_K]
    dist = jnp.sqrt(jnp.maximum(d2, 0.0) + 1e-12)
    src = idx.reshape(-1)
    dst = jnp.repeat(jnp.arange(n), _K)
    edge_index = jnp.stack([src, dst]).astype(jnp.int32)
    return edge_index, dist.reshape(-1)


# speedup vs baseline: 2.0763x; 2.0763x over previous
import jax
import jax.numpy as jnp
import numpy as np
from jax import lax
from jax.experimental import pallas as pl
from jax.experimental.pallas import tpu as pltpu
from jax.experimental.pallas import tpu_sc as plsc

_N = 4096
_K = 17
_KPAD = 32
_NC = 2
_NS = 16
_NW = _NC * _NS
_QPW = _N // _NW
_NBLK = _N // 16
_BIG = np.float32(1e30)


def _sc_body(kx_h, ky_h, kz_h, cell_h, oidx_h, od2_h,
             kx, ky, kz, cellv, d2, lvl1, oidxv, od2v):
    cid = lax.axis_index("c")
    sid = lax.axis_index("s")
    wid = sid * _NC + cid
    base = wid * _QPW
    pltpu.sync_copy(kx_h, kx)
    pltpu.sync_copy(ky_h, ky)
    pltpu.sync_copy(kz_h, kz)
    pltpu.sync_copy(cell_h, cellv)
    lanes = lax.iota(jnp.int32, 16)
    m0 = lanes == 0
    zf = jnp.zeros((16,), jnp.float32)
    zi = jnp.zeros((16,), jnp.int32)
    cx = cellv[pl.ds(0, 16)]
    cy = cellv[pl.ds(16, 16)]
    cz = cellv[pl.ds(32, 16)]
    one = jnp.float32(1.0)

    def per_query(q, _):
        i = base + q
        ti = i % _NBLK
        li = i // _NBLK
        qsplat = zi + (ti * 16 + li)
        qx = plsc.load_gather(kx, [qsplat])
        qy = plsc.load_gather(ky, [qsplat])
        qz = plsc.load_gather(kz, [qsplat])

        def over_sub(b, G):
            tb = b * 256
            accs = []
            for o in range(16):
                t = tb + o * 16
                dx = qx - kx[pl.ds(t, 16)]
                ax = jnp.abs(dx)
                wx = jnp.minimum(ax, one - ax) * cx
                acc = wx * wx
                dy = qy - ky[pl.ds(t, 16)]
                ay = jnp.abs(dy)
                wy = jnp.minimum(ay, one - ay) * cy
                acc = acc + wy * wy
                dz = qz - kz[pl.ds(t, 16)]
                az = jnp.abs(dz)
                wz = jnp.minimum(az, one - az) * cz
                acc = acc + wz * wz
                d2[pl.ds(t, 16)] = acc
                accs.append(acc)
            while len(accs) > 1:
                nxt = [jnp.minimum(accs[i], accs[i + 1])
                       for i in range(0, len(accs) - 1, 2)]
                if len(accs) % 2:
                    nxt.append(accs[-1])
                accs = nxt
            msub = accs[0]
            lvl1[pl.ds(b * 16, 16)] = msub
            return jnp.minimum(G, msub)

        G = lax.fori_loop(0, 16, over_sub, zf + _BIG)

        bi = ti // 16
        oi = ti % 16
        li_v = zi + li
        plsc.store_scatter(d2, [qsplat], zf + _BIG, mask=m0)
        vals = plsc.load_gather(d2, [(zi + bi * 16 + lanes) * 16 + li_v])
        mn = jnp.min(vals)
        sub = plsc.load_gather(lvl1, [lanes * 16 + li_v])
        subnew = jnp.where(lanes == bi, mn, sub)
        plsc.store_scatter(lvl1, [lanes * 16 + li_v], subnew)
        G = jnp.where(lanes == li_v, jnp.min(subnew), G)

        def extract(k, G):
            gv = jnp.min(G)
            lsel = plsc.all_reduce_ffs(G == gv)
            sub = plsc.load_gather(lvl1, [lanes * 16 + lsel])
            bsel = plsc.all_reduce_ffs(sub == gv)
            vals = plsc.load_gather(d2, [(bsel * 16 + lanes) * 16 + lsel])
            osel = plsc.all_reduce_ffs(vals == gv)
            e = lsel * 256 + bsel * 16 + osel
            slot = zi + q * _KPAD + k
            plsc.store_scatter(oidxv, [slot], e, mask=m0)
            plsc.store_scatter(od2v, [slot], zf + gv, mask=m0)
            plsc.store_scatter(d2, [(bsel * 16 + osel) * 16 + lsel],
                               zf + _BIG, mask=m0)
            vals2 = jnp.where(lanes == osel, _BIG, vals)
            mn = jnp.min(vals2)
            subnew = jnp.where(lanes == bsel, mn, sub)
            plsc.store_scatter(lvl1, [lanes * 16 + lsel], subnew)
            return jnp.where(lanes == lsel, jnp.min(subnew), G)

        lax.fori_loop(0, _K, extract, G)
        return 0

    lax.fori_loop(0, _QPW, per_query, 0)
    pltpu.sync_copy(oidxv, oidx_h.at[pl.ds(base * _KPAD, _QPW * _KPAD)])
    pltpu.sync_copy(od2v, od2_h.at[pl.ds(base * _KPAD, _QPW * _KPAD)])


def kernel(pos, cell):
    n = pos.shape[0]
    frac = pos / cell
    kx = frac[:, 0].reshape(16, _NBLK).T.reshape(-1)
    ky = frac[:, 1].reshape(16, _NBLK).T.reshape(-1)
    kz = frac[:, 2].reshape(16, _NBLK).T.reshape(-1)
    cellvec = jnp.repeat(cell, 16)

    f = pl.kernel(
        _sc_body,
        out_type=[
            jax.ShapeDtypeStruct((n * _KPAD,), jnp.int32),
            jax.ShapeDtypeStruct((n * _KPAD,), jnp.float32),
        ],
        mesh=plsc.VectorSubcoreMesh(core_axis_name="c", subcore_axis_name="s"),
        compiler_params=pltpu.CompilerParams(needs_layout_passes=False),
        scratch_types=[
            pltpu.VMEM((_N,), jnp.float32),
            pltpu.VMEM((_N,), jnp.float32),
            pltpu.VMEM((_N,), jnp.float32),
            pltpu.VMEM((48,), jnp.float32),
            pltpu.VMEM((_N,), jnp.float32),
            pltpu.VMEM((256,), jnp.float32),
            pltpu.VMEM((_QPW * _KPAD,), jnp.int32),
            pltpu.VMEM((_QPW * _KPAD,), jnp.float32),
        ],
    )
    oidx, od2 = f(kx, ky, kz, cellvec)

    idx = oidx.reshape(n, _KPAD)[:, :_K]
    d2 = od2.reshape(n, _KPAD)[:, :_K]
    dist = jnp.sqrt(jnp.maximum(d2, 0.0) + 1e-12)
    src = idx.reshape(-1)
    dst = jnp.repeat(jnp.arange(n), _K)
    edge_index = jnp.stack([src, dst]).astype(jnp.int32)
    return edge_index, dist.reshape(-1)
